# baseline (speedup 1.0000x reference)
import functools

import jax
import jax.numpy as jnp
from jax import lax
from jax.experimental import pallas as pl
from jax.experimental.pallas import tpu as pltpu
from jax.experimental.pallas import tpu_sc as plsc

D = 64
B = 1024
S = 200
N = B * S
EPS = 1e-5

NC = 2
NS = 16
NW = NC * NS
PER_W = N // NW
CH = 128
NCH = PER_W // CH


def _sc_gather(table, idx_flat):
    mesh = plsc.VectorSubcoreMesh(core_axis_name="c", subcore_axis_name="s")

    @functools.partial(
        pl.kernel,
        out_type=jax.ShapeDtypeStruct((N, D), jnp.float32),
        mesh=mesh,
        compiler_params=pltpu.CompilerParams(use_tc_tiling_on_sc=False),
        scratch_types=[
            pltpu.VMEM((PER_W,), jnp.int32),
            pltpu.VMEM((2, CH, D), jnp.float32),
            pltpu.SemaphoreType.DMA,
            pltpu.SemaphoreType.DMA,
        ],
    )
    def k(table_hbm, idx_hbm, out_hbm, idx_v, rows_v, gsem, osem):
        wid = lax.axis_index("s") * NC + lax.axis_index("c")
        base0 = wid * PER_W
        pltpu.sync_copy(idx_hbm.at[pl.ds(base0, PER_W)], idx_v)

        def body(c, _):
            slot = lax.rem(c, 2)
            g = pltpu.async_copy(
                table_hbm.at[idx_v.at[pl.ds(c * CH, CH)]],
                rows_v.at[slot], gsem)

            @pl.when(c >= 1)
            def _():
                pltpu.make_async_copy(
                    rows_v.at[1 - slot],
                    out_hbm.at[pl.ds(base0 + (c - 1) * CH, CH)],
                    osem).wait()

            g.wait()
            pltpu.async_copy(
                rows_v.at[slot],
                out_hbm.at[pl.ds(base0 + c * CH, CH)], osem)
            return ()

        lax.fori_loop(0, NCH, body, ())
        last = NCH - 1
        pltpu.make_async_copy(
            rows_v.at[lax.rem(last, 2)],
            out_hbm.at[pl.ds(base0 + last * CH, CH)], osem).wait()

    return k(table, idx_flat)


BB = 32


def _tc_ln_body(rows_ref, pos_ref, g_ref, b_ref, o_ref):
    e = rows_ref[...] + pos_ref[...][None]
    m = jnp.mean(e, axis=-1, keepdims=True)
    c = e - m
    v = jnp.mean(c * c, axis=-1, keepdims=True)
    o_ref[...] = (c * lax.rsqrt(v + EPS)) * g_ref[...][None] + b_ref[...][None]


def _tc_ln(rows3, pos, gamma, beta):
    return pl.pallas_call(
        _tc_ln_body,
        grid=(B // BB,),
        in_specs=[
            pl.BlockSpec((BB, S, D), lambda i: (i, 0, 0)),
            pl.BlockSpec((S, D), lambda i: (0, 0)),
            pl.BlockSpec((1, D), lambda i: (0, 0)),
            pl.BlockSpec((1, D), lambda i: (0, 0)),
        ],
        out_specs=pl.BlockSpec((BB, S, D), lambda i: (i, 0, 0)),
        out_shape=jax.ShapeDtypeStruct((B, S, D), jnp.float32),
    )(rows3, pos, gamma, beta)


def kernel(x, input_embedding_weight, position_embedding_weight, ln_gamma, ln_beta):
    idx_flat = x.astype(jnp.int32).reshape(N)
    rows = _sc_gather(input_embedding_weight, idx_flat)
    rows3 = rows.reshape(B, S, D)
    return _tc_ln(rows3, position_embedding_weight,
                  ln_gamma.reshape(1, D), ln_beta.reshape(1, D))
